# baseline (speedup 1.0000x reference)
import functools

import numpy as np

import jax
import jax.numpy as jnp
from jax import lax
from jax.experimental import pallas as pl
from jax.experimental.pallas import tpu as pltpu
from jax.experimental.pallas import tpu_sc as plsc

_N_TERMS = 64
_N_OUT = 4
_MASKS = [np.ones(_N_TERMS, dtype=bool) for _ in range(_N_OUT)]

_NUM_CORES = 2
_NUM_SUBCORES = 16
_NW = _NUM_CORES * _NUM_SUBCORES
_CHUNK = 400


def _masked_gather(thetas, cols):
    n, _ = thetas.shape
    w = int(cols.shape[0])
    rows_per = n // _NW
    mesh = plsc.VectorSubcoreMesh(core_axis_name="c", subcore_axis_name="s")

    n_chunks = n // _CHUNK

    @functools.partial(
        pl.kernel,
        out_type=jax.ShapeDtypeStruct((n, w), thetas.dtype),
        mesh=mesh,
        scratch_types=[
            pltpu.VMEM((_CHUNK, _N_TERMS), jnp.float32),
            pltpu.VMEM((_CHUNK, _N_TERMS), jnp.float32),
            pltpu.SemaphoreType.DMA,
            pltpu.SemaphoreType.DMA,
            pltpu.SemaphoreType.DMA,
            pltpu.SemaphoreType.DMA,
        ],
    )
    def k(x_hbm, o_hbm, buf0, buf1, si0, si1, so0, so1):
        wid = lax.axis_index("s") * _NUM_CORES + lax.axis_index("c")
        cnt = (n_chunks - wid + _NW - 1) // _NW

        def cbase(j):
            return pl.multiple_of((wid + j * _NW) * _CHUNK, 8)

        def start_in(j, buf, sem):
            pltpu.make_async_copy(
                x_hbm.at[pl.ds(cbase(j), _CHUNK), :], buf, sem).start()

        def wait_in(buf, sem):
            pltpu.make_async_copy(
                x_hbm.at[pl.ds(0, _CHUNK), :], buf, sem).wait()

        def start_out(j, buf, sem):
            pltpu.make_async_copy(
                buf, o_hbm.at[pl.ds(cbase(j), _CHUNK), :], sem).start()

        def wait_out(buf, sem):
            pltpu.make_async_copy(
                buf, o_hbm.at[pl.ds(0, _CHUNK), :], sem).wait()

        start_in(0, buf0, si0)

        def body(t, carry):
            a = 2 * t
            b = a + 1

            @pl.when(jnp.logical_and(b < cnt, t >= 1))
            def _():
                wait_out(buf1, so1)

            @pl.when(b < cnt)
            def _():
                start_in(b, buf1, si1)

            wait_in(buf0, si0)
            start_out(a, buf0, so0)

            @pl.when(b < cnt)
            def _():
                wait_in(buf1, si1)
                start_out(b, buf1, so1)

            @pl.when(a + 2 < cnt)
            def _():
                wait_out(buf0, so0)
                start_in(a + 2, buf0, si0)

            return carry

        lax.fori_loop(0, (cnt + 1) // 2, body, 0)
        wait_out(buf0, so0)
        wait_out(buf1, so1)

    return k(thetas)


def kernel(thetas, time_derivs, coeff_0, coeff_1, coeff_2, coeff_3):
    cols = np.nonzero(_MASKS[0])[0].astype(np.int32)
    gathered = _masked_gather(thetas, cols)
    sparse_thetas = (gathered,) * _N_OUT
    return sparse_thetas + (coeff_0, coeff_1, coeff_2, coeff_3)


# speedup vs baseline: 2.4047x; 2.4047x over previous
import numpy as np

import jax
import jax.numpy as jnp
from jax.experimental import pallas as pl

_N_TERMS = 64
_N_OUT = 4
_MASKS = [np.ones(_N_TERMS, dtype=bool) for _ in range(_N_OUT)]

_COL_BLOCK = 16384


def _gather_rows_kernel(x_ref, o_ref):
    o_ref[...] = x_ref[...]


def _masked_gather_t(thetas_t, rows):
    w, n = thetas_t.shape
    grid = (n + _COL_BLOCK - 1) // _COL_BLOCK
    return pl.pallas_call(
        _gather_rows_kernel,
        grid=(grid,),
        in_specs=[pl.BlockSpec((w, _COL_BLOCK), lambda i: (0, i))],
        out_specs=pl.BlockSpec((w, _COL_BLOCK), lambda i: (0, i)),
        out_shape=jax.ShapeDtypeStruct((w, n), thetas_t.dtype),
    )(thetas_t)


def kernel(thetas, time_derivs, coeff_0, coeff_1, coeff_2, coeff_3):
    rows = np.nonzero(_MASKS[0])[0].astype(np.int32)
    gathered = _masked_gather_t(thetas.T, rows).T
    sparse_thetas = (gathered,) * _N_OUT
    return sparse_thetas + (coeff_0, coeff_1, coeff_2, coeff_3)


# speedup vs baseline: 2.4159x; 1.0046x over previous
import numpy as np

import jax
import jax.numpy as jnp
from jax.experimental import pallas as pl

_N_TERMS = 64
_N_OUT = 4
_MASKS = [np.ones(_N_TERMS, dtype=bool) for _ in range(_N_OUT)]

_COL_BLOCK = 32768


def _gather_rows_kernel(x_ref, o_ref):
    o_ref[...] = x_ref[...]


def _masked_gather_t(thetas_t, rows):
    w, n = thetas_t.shape
    grid = (n + _COL_BLOCK - 1) // _COL_BLOCK
    return pl.pallas_call(
        _gather_rows_kernel,
        grid=(grid,),
        in_specs=[pl.BlockSpec((w, _COL_BLOCK), lambda i: (0, i))],
        out_specs=pl.BlockSpec((w, _COL_BLOCK), lambda i: (0, i)),
        out_shape=jax.ShapeDtypeStruct((w, n), thetas_t.dtype),
    )(thetas_t)


def kernel(thetas, time_derivs, coeff_0, coeff_1, coeff_2, coeff_3):
    rows = np.nonzero(_MASKS[0])[0].astype(np.int32)
    gathered = _masked_gather_t(thetas.T, rows).T
    sparse_thetas = (gathered,) * _N_OUT
    return sparse_thetas + (coeff_0, coeff_1, coeff_2, coeff_3)
